# baseline (speedup 1.0000x reference)
import math

import jax
import jax.numpy as jnp
from jax.experimental import pallas as pl


def _body(x_ref, wt_ref, b_ref, o_ref):
    C = x_ref.shape[2] // 4
    xm = x_ref[0]
    u = xm[:, 0:C] + xm[:, C:2 * C]
    v = xm[:, 2 * C:3 * C] + xm[:, 3 * C:]
    hq = u + v
    u3 = u.reshape(16, 2, 8, C)
    p2L = (u3[:, 0] + u3[:, 1]) * 0.25
    v3 = v.reshape(16, 2, 8, C)
    p2R = (v3[:, 0] + v3[:, 1]) * 0.25
    h3 = hq.reshape(8, 4, 8, C)
    p1 = (h3[:, 0] + h3[:, 1] + h3[:, 2] + h3[:, 3]) * 0.0625

    logits = jax.lax.dot_general(
        p1.reshape(64, C), wt_ref[...], (((1,), (0,)), ((), ())),
        preferred_element_type=jnp.float32) + b_ref[...]
    l0, l1, l2 = logits[:, 0:1], logits[:, 1:2], logits[:, 2:3]
    one = jnp.float32(1.0)
    zero = jnp.float32(0.0)
    m0 = jnp.where((l0 >= l1) & (l0 >= l2), one, zero)
    m1 = jnp.where((l1 > l0) & (l1 >= l2), one, zero)
    m2 = jnp.where((l2 > l0) & (l2 > l1), one, zero)

    p1r = p1.reshape(8, 2, 4, C)
    m0r = m0.reshape(8, 2, 4, 1)
    o1 = jnp.concatenate(
        [p1r[:, :, c] * m0r[:, :, c] for c in range(4)], axis=-1)
    o_ref[0, 0:16] = o1.reshape(16, 4 * C)

    m1g = m1.reshape(8, 4, 2, 1)
    m1e = m1g[:, :, 0].reshape(8, 1, 4, 1)
    m1o = m1g[:, :, 1].reshape(8, 1, 4, 1)
    Le = p2L.reshape(16, 4, 2, C)[:, :, 0].reshape(8, 2, 4, C)
    Lo = p2L.reshape(16, 4, 2, C)[:, :, 1].reshape(8, 2, 4, C)
    Re = p2R.reshape(16, 4, 2, C)[:, :, 0].reshape(8, 2, 4, C)
    Ro = p2R.reshape(16, 4, 2, C)[:, :, 1].reshape(8, 2, 4, C)
    o2 = jnp.concatenate([Le * m1e, Re * m1e, Lo * m1o, Ro * m1o], axis=-1)
    o_ref[0, 16:80] = o2.reshape(64, 4 * C)

    o3 = xm.reshape(8, 4, 8, 4 * C) * m2.reshape(8, 1, 8, 1)
    o_ref[0, 80:336] = o3.reshape(256, 4 * C)


def kernel(x, W_gate, b_gate, H, W):
    del H, W
    B, N, C = x.shape
    Hs = int(math.isqrt(N))
    Hr = Hs // 4
    n_out = Hr * Hr + (Hs // 2) * (Hs // 2) + N
    nm = n_out // 4
    xm = x.reshape(B, N // 4, 4 * C)
    wt = W_gate.T
    b2 = b_gate.reshape(1, -1)
    out = pl.pallas_call(
        _body,
        grid=(B,),
        in_specs=[
            pl.BlockSpec((1, N // 4, 4 * C), lambda b: (b, 0, 0)),
            pl.BlockSpec((C, W_gate.shape[0]), lambda b: (0, 0)),
            pl.BlockSpec((1, W_gate.shape[0]), lambda b: (0, 0)),
        ],
        out_specs=pl.BlockSpec((1, nm, 4 * C), lambda b: (b, 0, 0)),
        out_shape=jax.ShapeDtypeStruct((B, nm, 4 * C), x.dtype),
    )(xm, wt, b2)
    return out.reshape(B, n_out, C)


# speedup vs baseline: 3.3939x; 3.3939x over previous
import math

import numpy as np
import jax
import jax.numpy as jnp
from jax.experimental import pallas as pl


def _dot(a, b):
    return jax.lax.dot_general(a, b, (((1,), (0,)), ((), ())),
                               preferred_element_type=jnp.float32)


def _body(x_ref, a2_ref, a1_ref, wt_ref, b_ref, o_ref):
    C = x_ref.shape[2]
    N = x_ref.shape[1]
    Hr = int(math.isqrt(N)) // 4
    n2 = 4 * Hr * Hr
    n1 = Hr * Hr

    xs = x_ref[0]
    p2 = _dot(a2_ref[...], xs)
    p1 = _dot(a1_ref[...], p2)
    logits = _dot(p1, wt_ref[...]) + b_ref[...]
    l0, l1, l2 = logits[:, 0:1], logits[:, 1:2], logits[:, 2:3]
    one = jnp.float32(1.0)
    zero = jnp.float32(0.0)
    m0 = jnp.where((l0 >= l1) & (l0 >= l2), one, zero)
    m1 = jnp.where((l1 > l0) & (l1 >= l2), one, zero)
    m2 = jnp.where((l2 > l0) & (l2 > l1), one, zero)

    o_ref[0, 0:n1] = p1 * m0
    o2 = p2.reshape(Hr, 2, Hr, 2, C) * m1.reshape(Hr, 1, Hr, 1, 1)
    o_ref[0, n1:n1 + n2] = o2.reshape(n2, C)
    o3 = xs.reshape(Hr, 4, Hr, 4, C) * m2.reshape(Hr, 1, Hr, 1, 1)
    o_ref[0, n1 + n2:n1 + n2 + N] = o3.reshape(N, C)


def _pool_matrices(N):
    Hs = int(math.isqrt(N))
    H2 = Hs // 2
    a2 = np.zeros((H2 * H2, N), np.float32)
    for i in range(H2):
        for j in range(H2):
            for r in range(2):
                for c in range(2):
                    a2[i * H2 + j, (2 * i + r) * Hs + 2 * j + c] = 0.25
    H1 = Hs // 4
    a1 = np.zeros((H1 * H1, H2 * H2), np.float32)
    for i in range(H1):
        for j in range(H1):
            for r in range(2):
                for c in range(2):
                    a1[i * H1 + j, (2 * i + r) * H2 + 2 * j + c] = 0.25
    return a2, a1


def kernel(x, W_gate, b_gate, H, W):
    del H, W
    B, N, C = x.shape
    Hs = int(math.isqrt(N))
    n2 = (Hs // 2) * (Hs // 2)
    n1 = (Hs // 4) * (Hs // 4)
    n_out = n1 + n2 + N
    a2, a1 = _pool_matrices(N)
    wt = W_gate.T
    b2 = b_gate.reshape(1, -1)
    S = W_gate.shape[0]
    return pl.pallas_call(
        _body,
        grid=(B,),
        in_specs=[
            pl.BlockSpec((1, N, C), lambda b: (b, 0, 0)),
            pl.BlockSpec((n2, N), lambda b: (0, 0)),
            pl.BlockSpec((n1, n2), lambda b: (0, 0)),
            pl.BlockSpec((C, S), lambda b: (0, 0)),
            pl.BlockSpec((1, S), lambda b: (0, 0)),
        ],
        out_specs=pl.BlockSpec((1, n_out, C), lambda b: (b, 0, 0)),
        out_shape=jax.ShapeDtypeStruct((B, n_out, C), x.dtype),
    )(x, jnp.asarray(a2), jnp.asarray(a1), wt, b2)
